# Initial kernel scaffold; baseline (speedup 1.0000x reference)
import jax
import jax.numpy as jnp
from jax.experimental import pallas as pl


def kernel(x, W1, W2):
    raise NotImplementedError("write your pallas kernel here")


# speedup vs baseline: 2.1541x; 2.1541x over previous
import functools

import numpy as np
import jax
import jax.numpy as jnp
from jax import lax
from jax.experimental import pallas as pl
from jax.experimental.pallas import tpu as pltpu
from jax.experimental.pallas import tpu_sc as plsc

_BINS = 128
_INV_SQRT2 = 0.7071067811865476
_LOG2E = 1.4426950408889634


def _interp_matrix(n_in: int, n_out: int) -> np.ndarray:
    A = np.zeros((n_in, n_out), dtype=np.float64)
    scale = n_in / n_out
    for i in range(n_out):
        u = (i + 0.5) * scale - 0.5
        k0 = int(np.floor(u))
        f = u - k0
        k0c = min(max(k0, 0), n_in - 1)
        k1c = min(max(k0 + 1, 0), n_in - 1)
        A[k0c, i] += 1.0 - f
        A[k1c, i] += f
    return A


_A = _interp_matrix(4, 64)
_S = np.arange(64)
_XS, _YS, _ZS = _S // 16, (_S // 4) % 4, _S % 4
_KDT = np.ascontiguousarray(_A[_XS, :].T).astype(np.float32)
_KHW = np.einsum('sh,sw->shw', _A[_YS, :], _A[_ZS, :]).reshape(64, 4096).astype(np.float32)
_BD = (_S[:, None] // 16 == np.arange(4)[None, :]).astype(np.float32)
_HH, _WW = np.divmod(np.arange(4096), 64)
_BHW = (((_HH // 16) * 4 + (_WW // 16))[:, None]
        == np.arange(16)[None, :]).astype(np.float32)


def _pool_body(x_ref, bd_ref, bhw_ref, o_ref):
    a = x_ref[0]
    s1 = lax.dot_general(bd_ref[...], a, (((0,), (0,)), ((), ())),
                         preferred_element_type=jnp.float32)
    s2 = jnp.dot(s1, bhw_ref[...],
                 preferred_element_type=jnp.float32)
    o_ref[0] = s2 * (1.0 / 4096.0)


def _pool_call(x3):
    return pl.pallas_call(
        _pool_body,
        grid=(64,),
        in_specs=[
            pl.BlockSpec((1, 64, 4096), lambda i: (i, 0, 0)),
            pl.BlockSpec((64, 4), lambda i: (0, 0)),
            pl.BlockSpec((4096, 16), lambda i: (0, 0)),
        ],
        out_specs=pl.BlockSpec((1, 4, 16), lambda i: (i, 0, 0)),
        out_shape=jax.ShapeDtypeStruct((64, 4, 16), jnp.float32),
    )(x3, jnp.asarray(_BD), jnp.asarray(_BHW))


def _mlp_minmax_body(x_ref, p_ref, w1_ref, w2_ref, kdt_ref, khw_ref,
                     sw_ref, mn_ref, mx_ref):
    i = pl.program_id(0)

    @pl.when(i == 0)
    def _():
        for b in range(2):
            pb = p_ref[pl.ds(b * 32, 32), :]
            h = lax.dot_general(w1_ref[...], pb, (((1,), (0,)), ((), ())),
                                preferred_element_type=jnp.float32)
            mu = jnp.mean(h, axis=1, keepdims=True)
            var = jnp.mean((h - mu) * (h - mu), axis=1, keepdims=True)
            hn = (h - mu) * lax.rsqrt(var + 1e-5)
            g = 0.5 * hn * (1.0 + lax.erf(hn * _INV_SQRT2))
            o = lax.dot_general(w2_ref[...], g, (((1,), (0,)), ((), ())),
                                preferred_element_type=jnp.float32)
            swv = 1.0 / (1.0 + jnp.exp(-o))
            lhs = kdt_ref[...] * swv
            sw_ref[b] = jnp.dot(lhs, khw_ref[...],
                                preferred_element_type=jnp.float32)
        mn_ref[...] = jnp.full((1, 128), jnp.inf, jnp.float32)
        mx_ref[...] = jnp.full((1, 128), -jnp.inf, jnp.float32)

    b = i // 32
    swb = jnp.where(b == 0, sw_ref[0], sw_ref[1])
    w = x_ref[0] * swb
    mn_ref[...] = jnp.minimum(mn_ref[...], jnp.min(w))
    mx_ref[...] = jnp.maximum(mx_ref[...], jnp.max(w))


def _mlp_minmax_call(x3, pool2, W1, W2):
    return pl.pallas_call(
        _mlp_minmax_body,
        grid=(64,),
        in_specs=[
            pl.BlockSpec((1, 64, 4096), lambda i: (i, 0, 0)),
            pl.BlockSpec((64, 64), lambda i: (0, 0)),
            pl.BlockSpec((4, 32), lambda i: (0, 0)),
            pl.BlockSpec((1, 4), lambda i: (0, 0)),
            pl.BlockSpec((64, 64), lambda i: (0, 0)),
            pl.BlockSpec((64, 4096), lambda i: (0, 0)),
        ],
        out_specs=[
            pl.BlockSpec((2, 64, 4096), lambda i: (0, 0, 0)),
            pl.BlockSpec((1, 128), lambda i: (0, 0)),
            pl.BlockSpec((1, 128), lambda i: (0, 0)),
        ],
        out_shape=[
            jax.ShapeDtypeStruct((2, 64, 4096), jnp.float32),
            jax.ShapeDtypeStruct((1, 128), jnp.float32),
            jax.ShapeDtypeStruct((1, 128), jnp.float32),
        ],
    )(x3, pool2, W1, W2, jnp.asarray(_KDT), jnp.asarray(_KHW))


_CHUNK = 4 * 4096


def _sc_hist_body(x_hbm, sw_hbm, mm_hbm, out_hbm,
                  xb0, xb1, swb, mmb, hist, tileh, sem0, sem1):
    wid = lax.axis_index("s") * 2 + lax.axis_index("c")
    b = wid // 16
    d0 = (wid % 16) * 4

    pltpu.sync_copy(mm_hbm, mmb)
    gmin = mmb[pl.ds(0, 16)]
    gmax = mmb[pl.ds(128, 16)]
    scale = 128.0 / (gmax - gmin + 1e-8)
    gms = gmin * scale

    pltpu.sync_copy(sw_hbm.at[pl.ds(b * 262144 + d0 * 4096, _CHUNK)], swb)

    def _premul(j, carry):
        swb[pl.ds(j * 16, 16)] = swb[pl.ds(j * 16, 16)] * scale
        return carry
    lax.fori_loop(0, 1024, _premul, 0)

    zz = jnp.zeros((16,), jnp.float32)

    def _zh(j, carry):
        hist[pl.ds(j * 16, 16)] = zz
        return carry
    lax.fori_loop(0, 128, _zh, 0)

    laneoff = lax.iota(jnp.int32, 16) * 128
    ones = jnp.ones((16,), jnp.float32)

    def _xoff(c):
        return ((b * 32 + c) * 64 + d0) * 4096

    bufs = (xb0, xb1)
    sems = (sem0, sem1)
    copies = [None, None]
    copies[0] = pltpu.async_copy(x_hbm.at[pl.ds(_xoff(0), _CHUNK)], xb0, sem0)
    for c in range(32):
        pb = c % 2
        if c + 1 < 32:
            nb = (c + 1) % 2
            copies[nb] = pltpu.async_copy(
                x_hbm.at[pl.ds(_xoff(c + 1), _CHUNK)], bufs[nb], sems[nb])
        copies[pb].wait()
        buf = bufs[pb]

        def _inner(j, carry):
            xv = buf[pl.ds(j * 16, 16)]
            sv = swb[pl.ds(j * 16, 16)]
            bf = xv * sv - gms
            bi = bf.astype(jnp.int32)
            bi = jnp.minimum(jnp.maximum(bi, 0), 127) + laneoff
            plsc.addupdate_scatter(hist, [bi], ones)
            return carry
        lax.fori_loop(0, 1024, _inner, 0)

    for g in range(8):
        acc = hist[pl.ds(g * 16, 16)]
        for lane in range(1, 16):
            acc = acc + hist[pl.ds(lane * 128 + g * 16, 16)]
        tileh[pl.ds(g * 16, 16)] = acc
    pltpu.sync_copy(tileh, out_hbm.at[pl.ds(wid * 128, 128)])


def _sc_hist_call(x_flat, sw_flat, mm_flat):
    mesh = plsc.VectorSubcoreMesh(core_axis_name="c", subcore_axis_name="s")
    fn = pl.kernel(
        _sc_hist_body,
        mesh=mesh,
        compiler_params=pltpu.CompilerParams(needs_layout_passes=False),
        out_type=jax.ShapeDtypeStruct((4096,), jnp.float32),
        scratch_types=[
            pltpu.VMEM((_CHUNK,), jnp.float32),
            pltpu.VMEM((_CHUNK,), jnp.float32),
            pltpu.VMEM((_CHUNK,), jnp.float32),
            pltpu.VMEM((256,), jnp.float32),
            pltpu.VMEM((2048,), jnp.float32),
            pltpu.VMEM((128,), jnp.float32),
            pltpu.SemaphoreType.DMA,
            pltpu.SemaphoreType.DMA,
        ],
    )
    return fn(x_flat, sw_flat, mm_flat)


def _entropy_body(ph_ref, o_ref):
    h = jnp.sum(ph_ref[...], axis=0, keepdims=True)
    total = jnp.sum(h)
    prob = h / (total + 1e-10)
    ent = -jnp.sum(prob * jnp.log(prob + 1e-10)) * _LOG2E
    o_ref[...] = jnp.full((1, 1), ent, jnp.float32)


def _entropy_call(ph):
    return pl.pallas_call(
        _entropy_body,
        out_shape=jax.ShapeDtypeStruct((1, 1), jnp.float32),
    )(ph)


def kernel(x, W1, W2):
    x3 = x.reshape(64, 64, 4096)
    pool_p = _pool_call(x3)
    pool2 = pool_p.reshape(64, 64)
    sw, mn, mx = _mlp_minmax_call(x3, pool2, W1, W2)
    mm = jnp.concatenate([mn, mx], axis=0).reshape(256)
    ph = _sc_hist_call(x.reshape(-1), sw.reshape(-1), mm)
    ent = _entropy_call(ph.reshape(32, 128))
    return ent[0, 0]


# speedup vs baseline: 2.2673x; 1.0525x over previous
import functools

import numpy as np
import jax
import jax.numpy as jnp
from jax import lax
from jax.experimental import pallas as pl
from jax.experimental.pallas import tpu as pltpu
from jax.experimental.pallas import tpu_sc as plsc

_BINS = 128
_INV_SQRT2 = 0.7071067811865476
_LOG2E = 1.4426950408889634


def _interp_matrix(n_in: int, n_out: int) -> np.ndarray:
    A = np.zeros((n_in, n_out), dtype=np.float64)
    scale = n_in / n_out
    for i in range(n_out):
        u = (i + 0.5) * scale - 0.5
        k0 = int(np.floor(u))
        f = u - k0
        k0c = min(max(k0, 0), n_in - 1)
        k1c = min(max(k0 + 1, 0), n_in - 1)
        A[k0c, i] += 1.0 - f
        A[k1c, i] += f
    return A


_A = _interp_matrix(4, 64)
_S = np.arange(64)
_XS, _YS, _ZS = _S // 16, (_S // 4) % 4, _S % 4
_KDT = np.ascontiguousarray(_A[_XS, :].T).astype(np.float32)
_KHW = np.einsum('sh,sw->shw', _A[_YS, :], _A[_ZS, :]).reshape(64, 4096).astype(np.float32)
_BD = (_S[:, None] // 16 == np.arange(4)[None, :]).astype(np.float32)
_HH, _WW = np.divmod(np.arange(4096), 64)
_BHW = (((_HH // 16) * 4 + (_WW // 16))[:, None]
        == np.arange(16)[None, :]).astype(np.float32)


def _pool_body(x_ref, bd_ref, bhw_ref, o_ref):
    a = x_ref[0]
    s1 = lax.dot_general(bd_ref[...], a, (((0,), (0,)), ((), ())),
                         preferred_element_type=jnp.float32)
    s2 = jnp.dot(s1, bhw_ref[...],
                 preferred_element_type=jnp.float32)
    o_ref[0] = s2 * (1.0 / 4096.0)


def _pool_call(x3):
    return pl.pallas_call(
        _pool_body,
        grid=(64,),
        in_specs=[
            pl.BlockSpec((1, 64, 4096), lambda i: (i, 0, 0)),
            pl.BlockSpec((64, 4), lambda i: (0, 0)),
            pl.BlockSpec((4096, 16), lambda i: (0, 0)),
        ],
        out_specs=pl.BlockSpec((1, 4, 16), lambda i: (i, 0, 0)),
        out_shape=jax.ShapeDtypeStruct((64, 4, 16), jnp.float32),
    )(x3, jnp.asarray(_BD), jnp.asarray(_BHW))


def _mlp_minmax_body(x_ref, p_ref, w1_ref, w2_ref, kdt_ref, khw_ref,
                     sw_ref, mn_ref, mx_ref):
    i = pl.program_id(0)

    @pl.when(i == 0)
    def _():
        for b in range(2):
            pb = p_ref[pl.ds(b * 32, 32), :]
            h = lax.dot_general(w1_ref[...], pb, (((1,), (0,)), ((), ())),
                                preferred_element_type=jnp.float32)
            mu = jnp.mean(h, axis=1, keepdims=True)
            var = jnp.mean((h - mu) * (h - mu), axis=1, keepdims=True)
            hn = (h - mu) * lax.rsqrt(var + 1e-5)
            g = 0.5 * hn * (1.0 + lax.erf(hn * _INV_SQRT2))
            o = lax.dot_general(w2_ref[...], g, (((1,), (0,)), ((), ())),
                                preferred_element_type=jnp.float32)
            swv = 1.0 / (1.0 + jnp.exp(-o))
            lhs = kdt_ref[...] * swv
            sw_ref[b] = jnp.dot(lhs, khw_ref[...],
                                preferred_element_type=jnp.float32)
        mn_ref[...] = jnp.full((1, 128), jnp.inf, jnp.float32)
        mx_ref[...] = jnp.full((1, 128), -jnp.inf, jnp.float32)

    b = i // 32
    swb = sw_ref[pl.ds(b, 1)][0]
    w = x_ref[0] * swb
    mn_ref[...] = jnp.minimum(mn_ref[...], jnp.min(w))
    mx_ref[...] = jnp.maximum(mx_ref[...], jnp.max(w))


def _mlp_minmax_call(x3, pool2, W1, W2):
    return pl.pallas_call(
        _mlp_minmax_body,
        grid=(64,),
        in_specs=[
            pl.BlockSpec((1, 64, 4096), lambda i: (i, 0, 0)),
            pl.BlockSpec((64, 64), lambda i: (0, 0)),
            pl.BlockSpec((4, 32), lambda i: (0, 0)),
            pl.BlockSpec((1, 4), lambda i: (0, 0)),
            pl.BlockSpec((64, 64), lambda i: (0, 0)),
            pl.BlockSpec((64, 4096), lambda i: (0, 0)),
        ],
        out_specs=[
            pl.BlockSpec((2, 64, 4096), lambda i: (0, 0, 0)),
            pl.BlockSpec((1, 128), lambda i: (0, 0)),
            pl.BlockSpec((1, 128), lambda i: (0, 0)),
        ],
        out_shape=[
            jax.ShapeDtypeStruct((2, 64, 4096), jnp.float32),
            jax.ShapeDtypeStruct((1, 128), jnp.float32),
            jax.ShapeDtypeStruct((1, 128), jnp.float32),
        ],
    )(x3, pool2, W1, W2, jnp.asarray(_KDT), jnp.asarray(_KHW))


_CHUNK = 4 * 4096


def _sc_hist_body(x_hbm, sw_hbm, mm_hbm, out_hbm,
                  xb0, xb1, swb, mmb, hist, tileh, sem0, sem1):
    wid = lax.axis_index("s") * 2 + lax.axis_index("c")
    b = wid // 16
    d0 = (wid % 16) * 4

    pltpu.sync_copy(mm_hbm, mmb)
    gmin = mmb[pl.ds(0, 16)]
    gmax = mmb[pl.ds(128, 16)]
    scale = 128.0 / (gmax - gmin + 1e-8)
    gms = gmin * scale

    pltpu.sync_copy(sw_hbm.at[pl.ds(b * 262144 + d0 * 4096, _CHUNK)], swb)

    def _premul(j, carry):
        base = j * 64
        for k in range(4):
            swb[pl.ds(base + k * 16, 16)] = swb[pl.ds(base + k * 16, 16)] * scale
        return carry
    lax.fori_loop(0, 256, _premul, 0)

    zz = jnp.zeros((16,), jnp.float32)

    def _zh(j, carry):
        hist[pl.ds(j * 16, 16)] = zz
        return carry
    lax.fori_loop(0, 128, _zh, 0)

    laneoff = lax.iota(jnp.int32, 16) * 128
    aoff = laneoff.astype(jnp.float32) - gms
    clampv = laneoff + 127
    ones = jnp.ones((16,), jnp.float32)

    def _xoff(c):
        return ((b * 32 + c) * 64 + d0) * 4096

    bufs = (xb0, xb1)
    sems = (sem0, sem1)
    copies = [None, None]
    copies[0] = pltpu.async_copy(x_hbm.at[pl.ds(_xoff(0), _CHUNK)], xb0, sem0)
    for c in range(32):
        pb = c % 2
        if c + 1 < 32:
            nb = (c + 1) % 2
            copies[nb] = pltpu.async_copy(
                x_hbm.at[pl.ds(_xoff(c + 1), _CHUNK)], bufs[nb], sems[nb])
        copies[pb].wait()
        buf = bufs[pb]

        def _inner(j, carry):
            base = j * 128
            for k in range(8):
                off = base + k * 16
                xv = buf[pl.ds(off, 16)]
                sv = swb[pl.ds(off, 16)]
                bi = jnp.minimum((xv * sv + aoff).astype(jnp.int32), clampv)
                plsc.addupdate_scatter(hist, [bi], ones)
            return carry
        lax.fori_loop(0, 128, _inner, 0)

    for g in range(8):
        acc = hist[pl.ds(g * 16, 16)]
        for lane in range(1, 16):
            acc = acc + hist[pl.ds(lane * 128 + g * 16, 16)]
        tileh[pl.ds(g * 16, 16)] = acc
    pltpu.sync_copy(tileh, out_hbm.at[pl.ds(wid * 128, 128)])


def _sc_hist_call(x_flat, sw_flat, mm_flat):
    mesh = plsc.VectorSubcoreMesh(core_axis_name="c", subcore_axis_name="s")
    fn = pl.kernel(
        _sc_hist_body,
        mesh=mesh,
        compiler_params=pltpu.CompilerParams(needs_layout_passes=False),
        out_type=jax.ShapeDtypeStruct((4096,), jnp.float32),
        scratch_types=[
            pltpu.VMEM((_CHUNK,), jnp.float32),
            pltpu.VMEM((_CHUNK,), jnp.float32),
            pltpu.VMEM((_CHUNK,), jnp.float32),
            pltpu.VMEM((256,), jnp.float32),
            pltpu.VMEM((2048,), jnp.float32),
            pltpu.VMEM((128,), jnp.float32),
            pltpu.SemaphoreType.DMA,
            pltpu.SemaphoreType.DMA,
        ],
    )
    return fn(x_flat, sw_flat, mm_flat)


def _entropy_body(ph_ref, o_ref):
    h = jnp.sum(ph_ref[...], axis=0, keepdims=True)
    total = jnp.sum(h)
    prob = h / (total + 1e-10)
    ent = -jnp.sum(prob * jnp.log(prob + 1e-10)) * _LOG2E
    o_ref[...] = jnp.full((1, 1), ent, jnp.float32)


def _entropy_call(ph):
    return pl.pallas_call(
        _entropy_body,
        out_shape=jax.ShapeDtypeStruct((1, 1), jnp.float32),
    )(ph)


def kernel(x, W1, W2):
    x3 = x.reshape(64, 64, 4096)
    pool_p = _pool_call(x3)
    pool2 = pool_p.reshape(64, 64)
    sw, mn, mx = _mlp_minmax_call(x3, pool2, W1, W2)
    mm = jnp.concatenate([mn, mx], axis=0).reshape(256)
    ph = _sc_hist_call(x.reshape(-1), sw.reshape(-1), mm)
    ent = _entropy_call(ph.reshape(32, 128))
    return ent[0, 0]


# speedup vs baseline: 4.0753x; 1.7975x over previous
import functools

import numpy as np
import jax
import jax.numpy as jnp
from jax import lax
from jax.experimental import pallas as pl
from jax.experimental.pallas import tpu as pltpu
from jax.experimental.pallas import tpu_sc as plsc

_BINS = 128
_INV_SQRT2 = 0.7071067811865476
_LOG2E = 1.4426950408889634


def _interp_matrix(n_in: int, n_out: int) -> np.ndarray:
    A = np.zeros((n_in, n_out), dtype=np.float64)
    scale = n_in / n_out
    for i in range(n_out):
        u = (i + 0.5) * scale - 0.5
        k0 = int(np.floor(u))
        f = u - k0
        k0c = min(max(k0, 0), n_in - 1)
        k1c = min(max(k0 + 1, 0), n_in - 1)
        A[k0c, i] += 1.0 - f
        A[k1c, i] += f
    return A


_A = _interp_matrix(4, 64)
_S = np.arange(64)
_XS, _YS, _ZS = _S // 16, (_S // 4) % 4, _S % 4
_KDT = np.ascontiguousarray(_A[_XS, :].T).astype(np.float32)
_KHW = np.einsum('sh,sw->shw', _A[_YS, :], _A[_ZS, :]).reshape(64, 4096).astype(np.float32)
_BD = (_S[:, None] // 16 == np.arange(4)[None, :]).astype(np.float32)
_HH, _WW = np.divmod(np.arange(4096), 64)
_BHW = (((_HH // 16) * 4 + (_WW // 16))[:, None]
        == np.arange(16)[None, :]).astype(np.float32)


def _pool_body(x_ref, bd_ref, bhw_ref, o_ref):
    a = x_ref[0]
    s1 = lax.dot_general(bd_ref[...], a, (((0,), (0,)), ((), ())),
                         preferred_element_type=jnp.float32)
    s2 = jnp.dot(s1, bhw_ref[...],
                 preferred_element_type=jnp.float32)
    o_ref[0] = s2 * (1.0 / 4096.0)


def _pool_call(x3):
    return pl.pallas_call(
        _pool_body,
        grid=(64,),
        in_specs=[
            pl.BlockSpec((1, 64, 4096), lambda i: (i, 0, 0)),
            pl.BlockSpec((64, 4), lambda i: (0, 0)),
            pl.BlockSpec((4096, 16), lambda i: (0, 0)),
        ],
        out_specs=pl.BlockSpec((1, 4, 16), lambda i: (i, 0, 0)),
        out_shape=jax.ShapeDtypeStruct((64, 4, 16), jnp.float32),
    )(x3, jnp.asarray(_BD), jnp.asarray(_BHW))


def _mlp_minmax_body(x_ref, p_ref, w1_ref, w2_ref, kdt_ref, khw_ref,
                     sw_ref, mn_ref, mx_ref):
    i = pl.program_id(0)

    @pl.when(i == 0)
    def _():
        for b in range(2):
            pb = p_ref[pl.ds(b * 32, 32), :]
            h = lax.dot_general(w1_ref[...], pb, (((1,), (0,)), ((), ())),
                                preferred_element_type=jnp.float32)
            mu = jnp.mean(h, axis=1, keepdims=True)
            var = jnp.mean((h - mu) * (h - mu), axis=1, keepdims=True)
            hn = (h - mu) * lax.rsqrt(var + 1e-5)
            g = 0.5 * hn * (1.0 + lax.erf(hn * _INV_SQRT2))
            o = lax.dot_general(w2_ref[...], g, (((1,), (0,)), ((), ())),
                                preferred_element_type=jnp.float32)
            swv = 1.0 / (1.0 + jnp.exp(-o))
            lhs = kdt_ref[...] * swv
            sw_ref[b] = jnp.dot(lhs, khw_ref[...],
                                preferred_element_type=jnp.float32)
        mn_ref[...] = jnp.full((1, 128), jnp.inf, jnp.float32)
        mx_ref[...] = jnp.full((1, 128), -jnp.inf, jnp.float32)

    b = i // 32
    swb = sw_ref[pl.ds(b, 1)][0]
    w = x_ref[0] * swb
    mn_ref[...] = jnp.minimum(mn_ref[...], jnp.min(w))
    mx_ref[...] = jnp.maximum(mx_ref[...], jnp.max(w))


def _mlp_minmax_call(x3, pool2, W1, W2):
    return pl.pallas_call(
        _mlp_minmax_body,
        grid=(64,),
        in_specs=[
            pl.BlockSpec((1, 64, 4096), lambda i: (i, 0, 0)),
            pl.BlockSpec((64, 64), lambda i: (0, 0)),
            pl.BlockSpec((4, 32), lambda i: (0, 0)),
            pl.BlockSpec((1, 4), lambda i: (0, 0)),
            pl.BlockSpec((64, 64), lambda i: (0, 0)),
            pl.BlockSpec((64, 4096), lambda i: (0, 0)),
        ],
        out_specs=[
            pl.BlockSpec((2, 64, 4096), lambda i: (0, 0, 0)),
            pl.BlockSpec((1, 128), lambda i: (0, 0)),
            pl.BlockSpec((1, 128), lambda i: (0, 0)),
        ],
        out_shape=[
            jax.ShapeDtypeStruct((2, 64, 4096), jnp.float32),
            jax.ShapeDtypeStruct((1, 128), jnp.float32),
            jax.ShapeDtypeStruct((1, 128), jnp.float32),
        ],
    )(x3, pool2, W1, W2, jnp.asarray(_KDT), jnp.asarray(_KHW))


_CHUNK = 4 * 4096


def _sc_hist_body(x_hbm, sw_hbm, mm_hbm, out_hbm,
                  xb0, xb1, swb, mmb, hist, tileh, sem0, sem1):
    wid = lax.axis_index("s") * 2 + lax.axis_index("c")
    b = wid // 16
    d0 = (wid % 16) * 4

    pltpu.sync_copy(mm_hbm, mmb)
    gmin = mmb[pl.ds(0, 16)]
    gmax = mmb[pl.ds(128, 16)]
    scale = 128.0 / (gmax - gmin + 1e-8)
    gms = gmin * scale

    pltpu.sync_copy(sw_hbm.at[pl.ds(b * 262144 + d0 * 4096, _CHUNK)], swb)

    @plsc.parallel_loop(0, 1024, unroll=4)
    def _premul(j):
        swb[pl.ds(j * 16, 16)] = swb[pl.ds(j * 16, 16)] * scale

    zz = jnp.zeros((16,), jnp.float32)

    def _zh(j, carry):
        hist[pl.ds(j * 16, 16)] = zz
        return carry
    lax.fori_loop(0, 128, _zh, 0)

    laneoff = lax.iota(jnp.int32, 16) * 128
    aoff = laneoff.astype(jnp.float32) - gms
    clampv = laneoff + 127
    ones = jnp.ones((16,), jnp.float32)

    def _xoff(c):
        return ((b * 32 + c) * 64 + d0) * 4096

    bufs = (xb0, xb1)
    sems = (sem0, sem1)
    copies = [None, None]
    copies[0] = pltpu.async_copy(x_hbm.at[pl.ds(_xoff(0), _CHUNK)], xb0, sem0)
    for c in range(32):
        pb = c % 2
        if c + 1 < 32:
            nb = (c + 1) % 2
            copies[nb] = pltpu.async_copy(
                x_hbm.at[pl.ds(_xoff(c + 1), _CHUNK)], bufs[nb], sems[nb])
        copies[pb].wait()
        buf = bufs[pb]

        @plsc.parallel_loop(0, 1024, unroll=8)
        def _inner(j):
            off = j * 16
            xv = buf[pl.ds(off, 16)]
            sv = swb[pl.ds(off, 16)]
            bi = jnp.minimum((xv * sv + aoff).astype(jnp.int32), clampv)
            plsc.addupdate_scatter(hist, [bi], ones)

    for g in range(8):
        acc = hist[pl.ds(g * 16, 16)]
        for lane in range(1, 16):
            acc = acc + hist[pl.ds(lane * 128 + g * 16, 16)]
        tileh[pl.ds(g * 16, 16)] = acc
    pltpu.sync_copy(tileh, out_hbm.at[pl.ds(wid * 128, 128)])


def _sc_hist_call(x_flat, sw_flat, mm_flat):
    mesh = plsc.VectorSubcoreMesh(core_axis_name="c", subcore_axis_name="s")
    fn = pl.kernel(
        _sc_hist_body,
        mesh=mesh,
        compiler_params=pltpu.CompilerParams(needs_layout_passes=False),
        out_type=jax.ShapeDtypeStruct((4096,), jnp.float32),
        scratch_types=[
            pltpu.VMEM((_CHUNK,), jnp.float32),
            pltpu.VMEM((_CHUNK,), jnp.float32),
            pltpu.VMEM((_CHUNK,), jnp.float32),
            pltpu.VMEM((256,), jnp.float32),
            pltpu.VMEM((2048,), jnp.float32),
            pltpu.VMEM((128,), jnp.float32),
            pltpu.SemaphoreType.DMA,
            pltpu.SemaphoreType.DMA,
        ],
    )
    return fn(x_flat, sw_flat, mm_flat)


def _entropy_body(ph_ref, o_ref):
    h = jnp.sum(ph_ref[...], axis=0, keepdims=True)
    total = jnp.sum(h)
    prob = h / (total + 1e-10)
    ent = -jnp.sum(prob * jnp.log(prob + 1e-10)) * _LOG2E
    o_ref[...] = jnp.full((1, 1), ent, jnp.float32)


def _entropy_call(ph):
    return pl.pallas_call(
        _entropy_body,
        out_shape=jax.ShapeDtypeStruct((1, 1), jnp.float32),
    )(ph)


def kernel(x, W1, W2):
    x3 = x.reshape(64, 64, 4096)
    pool_p = _pool_call(x3)
    pool2 = pool_p.reshape(64, 64)
    sw, mn, mx = _mlp_minmax_call(x3, pool2, W1, W2)
    mm = jnp.concatenate([mn, mx], axis=0).reshape(256)
    ph = _sc_hist_call(x.reshape(-1), sw.reshape(-1), mm)
    ent = _entropy_call(ph.reshape(32, 128))
    return ent[0, 0]


# speedup vs baseline: 4.1057x; 1.0075x over previous
import functools

import numpy as np
import jax
import jax.numpy as jnp
from jax import lax
from jax.experimental import pallas as pl
from jax.experimental.pallas import tpu as pltpu
from jax.experimental.pallas import tpu_sc as plsc

_BINS = 128
_INV_SQRT2 = 0.7071067811865476
_LOG2E = 1.4426950408889634


def _interp_matrix(n_in: int, n_out: int) -> np.ndarray:
    A = np.zeros((n_in, n_out), dtype=np.float64)
    scale = n_in / n_out
    for i in range(n_out):
        u = (i + 0.5) * scale - 0.5
        k0 = int(np.floor(u))
        f = u - k0
        k0c = min(max(k0, 0), n_in - 1)
        k1c = min(max(k0 + 1, 0), n_in - 1)
        A[k0c, i] += 1.0 - f
        A[k1c, i] += f
    return A


_A = _interp_matrix(4, 64)
_S = np.arange(64)
_XS, _YS, _ZS = _S // 16, (_S // 4) % 4, _S % 4

_R2048 = np.arange(2048)
_L128 = np.arange(128)
_RLANE = (((_L128 % 64) // 16)[:, None] == np.arange(4)[None, :]).astype(np.float32)
_LROW = ((( _R2048 // 512) * 4 + (_R2048 % 32) // 8)[:, None]
         == np.arange(16)[None, :]).astype(np.float32)
_D_OF_R = _R2048 // 32
_M0 = (_A[_XS][:, _D_OF_R] * _A[_YS][:, 2 * (_R2048 % 32)]).astype(np.float32)
_M1 = (_A[_XS][:, _D_OF_R] * _A[_YS][:, 2 * (_R2048 % 32) + 1]).astype(np.float32)
_KB0 = np.where((_L128 // 64 == 0)[None, :], _A[_ZS][:, _L128 % 64], 0.0).astype(np.float32)
_KB1 = np.where((_L128 // 64 == 1)[None, :], _A[_ZS][:, _L128 % 64], 0.0).astype(np.float32)


def _pool_body(x_ref, rl_ref, lr_ref, o_ref):
    a = x_ref[...]
    t1 = jnp.dot(a, rl_ref[...],
                 preferred_element_type=jnp.float32)
    t2 = lax.dot_general(lr_ref[...], t1, (((0,), (0,)), ((), ())),
                         preferred_element_type=jnp.float32)
    o_ref[0] = t2 * (1.0 / 4096.0)


def _pool_call(x2):
    return pl.pallas_call(
        _pool_body,
        grid=(64,),
        in_specs=[
            pl.BlockSpec((2048, 128), lambda i: (i, 0)),
            pl.BlockSpec((128, 4), lambda i: (0, 0)),
            pl.BlockSpec((2048, 16), lambda i: (0, 0)),
        ],
        out_specs=pl.BlockSpec((1, 16, 4), lambda i: (i, 0, 0)),
        out_shape=jax.ShapeDtypeStruct((64, 16, 4), jnp.float32),
    )(x2, jnp.asarray(_RLANE), jnp.asarray(_LROW))


def _mlp_minmax_body(x_ref, p_ref, w1_ref, w2_ref, m0_ref, m1_ref,
                     kb0_ref, kb1_ref, sw_ref, mn_ref, mx_ref):
    i = pl.program_id(0)

    @pl.when(i % 32 == 0)
    def _():
        pb = p_ref[0]
        h = lax.dot_general(w1_ref[...], pb, (((1,), (0,)), ((), ())),
                            preferred_element_type=jnp.float32)
        mu = jnp.mean(h, axis=1, keepdims=True)
        var = jnp.mean((h - mu) * (h - mu), axis=1, keepdims=True)
        hn = (h - mu) * lax.rsqrt(var + 1e-5)
        g = 0.5 * hn * (1.0 + lax.erf(hn * _INV_SQRT2))
        o = lax.dot_general(g, w2_ref[...], (((0,), (1,)), ((), ())),
                            preferred_element_type=jnp.float32)
        swcol = 1.0 / (1.0 + jnp.exp(-o))
        x0 = lax.dot_general(m0_ref[...] * swcol, kb0_ref[...],
                             (((0,), (0,)), ((), ())),
                             preferred_element_type=jnp.float32)
        x1 = lax.dot_general(m1_ref[...] * swcol, kb1_ref[...],
                             (((0,), (0,)), ((), ())),
                             preferred_element_type=jnp.float32)
        sw_ref[0] = x0 + x1

    @pl.when(i == 0)
    def _():
        mn_ref[...] = jnp.full((1, 128), jnp.inf, jnp.float32)
        mx_ref[...] = jnp.full((1, 128), -jnp.inf, jnp.float32)

    w = x_ref[...] * sw_ref[0]
    mn_ref[...] = jnp.minimum(mn_ref[...], jnp.min(w))
    mx_ref[...] = jnp.maximum(mx_ref[...], jnp.max(w))


def _mlp_minmax_call(x2, pool3, W1, W2):
    return pl.pallas_call(
        _mlp_minmax_body,
        grid=(64,),
        in_specs=[
            pl.BlockSpec((2048, 128), lambda i: (i, 0)),
            pl.BlockSpec((1, 32, 64), lambda i: (i // 32, 0, 0)),
            pl.BlockSpec((4, 32), lambda i: (0, 0)),
            pl.BlockSpec((1, 4), lambda i: (0, 0)),
            pl.BlockSpec((64, 2048), lambda i: (0, 0)),
            pl.BlockSpec((64, 2048), lambda i: (0, 0)),
            pl.BlockSpec((64, 128), lambda i: (0, 0)),
            pl.BlockSpec((64, 128), lambda i: (0, 0)),
        ],
        out_specs=[
            pl.BlockSpec((1, 2048, 128), lambda i: (i // 32, 0, 0)),
            pl.BlockSpec((1, 128), lambda i: (0, 0)),
            pl.BlockSpec((1, 128), lambda i: (0, 0)),
        ],
        out_shape=[
            jax.ShapeDtypeStruct((2, 2048, 128), jnp.float32),
            jax.ShapeDtypeStruct((1, 128), jnp.float32),
            jax.ShapeDtypeStruct((1, 128), jnp.float32),
        ],
    )(x2, pool3, W1, W2, jnp.asarray(_M0), jnp.asarray(_M1),
      jnp.asarray(_KB0), jnp.asarray(_KB1))


_CHUNK = 4 * 4096


def _sc_hist_body(x_hbm, sw_hbm, mm_hbm, out_hbm,
                  xb0, xb1, swb, mmb, hist, tileh, sem0, sem1):
    wid = lax.axis_index("s") * 2 + lax.axis_index("c")
    b = wid // 16
    d0 = (wid % 16) * 4

    pltpu.sync_copy(mm_hbm, mmb)
    gmin = mmb[pl.ds(0, 16)]
    gmax = mmb[pl.ds(128, 16)]
    scale = 128.0 / (gmax - gmin + 1e-8)
    gms = gmin * scale

    pltpu.sync_copy(sw_hbm.at[pl.ds(b * 262144 + d0 * 4096, _CHUNK)], swb)

    @plsc.parallel_loop(0, 1024, unroll=4)
    def _premul(j):
        swb[pl.ds(j * 16, 16)] = swb[pl.ds(j * 16, 16)] * scale

    zz = jnp.zeros((16,), jnp.float32)

    def _zh(j, carry):
        hist[pl.ds(j * 16, 16)] = zz
        return carry
    lax.fori_loop(0, 128, _zh, 0)

    laneoff = lax.iota(jnp.int32, 16) * 128
    aoff = laneoff.astype(jnp.float32) - gms
    clampv = laneoff + 127
    ones = jnp.ones((16,), jnp.float32)

    def _xoff(c):
        return ((b * 32 + c) * 64 + d0) * 4096

    bufs = (xb0, xb1)
    sems = (sem0, sem1)
    copies = [None, None]
    copies[0] = pltpu.async_copy(x_hbm.at[pl.ds(_xoff(0), _CHUNK)], xb0, sem0)
    for c in range(32):
        pb = c % 2
        if c + 1 < 32:
            nb = (c + 1) % 2
            copies[nb] = pltpu.async_copy(
                x_hbm.at[pl.ds(_xoff(c + 1), _CHUNK)], bufs[nb], sems[nb])
        copies[pb].wait()
        buf = bufs[pb]

        @plsc.parallel_loop(0, 1024, unroll=8)
        def _inner(j):
            off = j * 16
            xv = buf[pl.ds(off, 16)]
            sv = swb[pl.ds(off, 16)]
            bi = jnp.minimum((xv * sv + aoff).astype(jnp.int32), clampv)
            plsc.addupdate_scatter(hist, [bi], ones)

    for g in range(8):
        acc = hist[pl.ds(g * 16, 16)]
        for lane in range(1, 16):
            acc = acc + hist[pl.ds(lane * 128 + g * 16, 16)]
        tileh[pl.ds(g * 16, 16)] = acc
    pltpu.sync_copy(tileh, out_hbm.at[pl.ds(wid * 128, 128)])


def _sc_hist_call(x_flat, sw_flat, mm_flat):
    mesh = plsc.VectorSubcoreMesh(core_axis_name="c", subcore_axis_name="s")
    fn = pl.kernel(
        _sc_hist_body,
        mesh=mesh,
        compiler_params=pltpu.CompilerParams(needs_layout_passes=False),
        out_type=jax.ShapeDtypeStruct((4096,), jnp.float32),
        scratch_types=[
            pltpu.VMEM((_CHUNK,), jnp.float32),
            pltpu.VMEM((_CHUNK,), jnp.float32),
            pltpu.VMEM((_CHUNK,), jnp.float32),
            pltpu.VMEM((256,), jnp.float32),
            pltpu.VMEM((2048,), jnp.float32),
            pltpu.VMEM((128,), jnp.float32),
            pltpu.SemaphoreType.DMA,
            pltpu.SemaphoreType.DMA,
        ],
    )
    return fn(x_flat, sw_flat, mm_flat)


def _entropy_body(ph_ref, o_ref):
    h = jnp.sum(ph_ref[...], axis=0, keepdims=True)
    total = jnp.sum(h)
    prob = h / (total + 1e-10)
    ent = -jnp.sum(prob * jnp.log(prob + 1e-10)) * _LOG2E
    o_ref[...] = jnp.full((1, 1), ent, jnp.float32)


def _entropy_call(ph):
    return pl.pallas_call(
        _entropy_body,
        out_shape=jax.ShapeDtypeStruct((1, 1), jnp.float32),
    )(ph)


def kernel(x, W1, W2):
    x1 = x.reshape(-1)
    x2 = x1.reshape(131072, 128)
    pool_p = _pool_call(x2)
    pool3 = pool_p.reshape(2, 32, 64)
    sw, mn, mx = _mlp_minmax_call(x2, pool3, W1, W2)
    mm = jnp.concatenate([mn, mx], axis=0).reshape(256)
    ph = _sc_hist_call(x1, sw.reshape(-1), mm)
    ent = _entropy_call(ph.reshape(32, 128))
    return ent[0, 0]


# speedup vs baseline: 5.3434x; 1.3014x over previous
import functools

import numpy as np
import jax
import jax.numpy as jnp
from jax import lax
from jax.experimental import pallas as pl
from jax.experimental.pallas import tpu as pltpu
from jax.experimental.pallas import tpu_sc as plsc

_BINS = 128
_INV_SQRT2 = 0.7071067811865476
_LOG2E = 1.4426950408889634


def _interp_matrix(n_in: int, n_out: int) -> np.ndarray:
    A = np.zeros((n_in, n_out), dtype=np.float64)
    scale = n_in / n_out
    for i in range(n_out):
        u = (i + 0.5) * scale - 0.5
        k0 = int(np.floor(u))
        f = u - k0
        k0c = min(max(k0, 0), n_in - 1)
        k1c = min(max(k0 + 1, 0), n_in - 1)
        A[k0c, i] += 1.0 - f
        A[k1c, i] += f
    return A


_A = _interp_matrix(4, 64)
_S = np.arange(64)
_XS, _YS, _ZS = _S // 16, (_S // 4) % 4, _S % 4

_R2048 = np.arange(2048)
_L128 = np.arange(128)
_RLANE = (((_L128 % 64) // 16)[:, None] == np.arange(4)[None, :]).astype(np.float32)
_LROW = ((( _R2048 // 512) * 4 + (_R2048 % 32) // 8)[:, None]
         == np.arange(16)[None, :]).astype(np.float32)
_D_OF_R = _R2048 // 32
_M0 = (_A[_XS][:, _D_OF_R] * _A[_YS][:, 2 * (_R2048 % 32)]).astype(np.float32)
_M1 = (_A[_XS][:, _D_OF_R] * _A[_YS][:, 2 * (_R2048 % 32) + 1]).astype(np.float32)
_KB0 = np.where((_L128 // 64 == 0)[None, :], _A[_ZS][:, _L128 % 64], 0.0).astype(np.float32)
_KB1 = np.where((_L128 // 64 == 1)[None, :], _A[_ZS][:, _L128 % 64], 0.0).astype(np.float32)


def _pool_body(x_ref, rl_ref, lr_ref, o_ref):
    a = x_ref[...].reshape(2048, 128)
    t1 = jnp.dot(a, rl_ref[...],
                 preferred_element_type=jnp.float32)
    t2 = lax.dot_general(lr_ref[...], t1, (((0,), (0,)), ((), ())),
                         preferred_element_type=jnp.float32)
    o_ref[0] = t2 * (1.0 / 4096.0)


def _pool_call(x1):
    return pl.pallas_call(
        _pool_body,
        grid=(64,),
        in_specs=[
            pl.BlockSpec((262144,), lambda i: (i,)),
            pl.BlockSpec((128, 4), lambda i: (0, 0)),
            pl.BlockSpec((2048, 16), lambda i: (0, 0)),
        ],
        out_specs=pl.BlockSpec((1, 16, 4), lambda i: (i, 0, 0)),
        out_shape=jax.ShapeDtypeStruct((64, 16, 4), jnp.float32),
    )(x1, jnp.asarray(_RLANE), jnp.asarray(_LROW))


def _mlp_minmax_body(x_ref, p_ref, w1_ref, w2_ref, m0_ref, m1_ref,
                     kb0_ref, kb1_ref, sw_ref, mn_ref, mx_ref):
    i = pl.program_id(0)

    @pl.when(i % 32 == 0)
    def _():
        pb = p_ref[0]
        h = lax.dot_general(w1_ref[...], pb, (((1,), (0,)), ((), ())),
                            preferred_element_type=jnp.float32)
        mu = jnp.mean(h, axis=1, keepdims=True)
        var = jnp.mean((h - mu) * (h - mu), axis=1, keepdims=True)
        hn = (h - mu) * lax.rsqrt(var + 1e-5)
        g = 0.5 * hn * (1.0 + lax.erf(hn * _INV_SQRT2))
        o = lax.dot_general(g, w2_ref[...], (((0,), (1,)), ((), ())),
                            preferred_element_type=jnp.float32)
        swcol = 1.0 / (1.0 + jnp.exp(-o))
        x0 = lax.dot_general(m0_ref[...] * swcol, kb0_ref[...],
                             (((0,), (0,)), ((), ())),
                             preferred_element_type=jnp.float32)
        x1 = lax.dot_general(m1_ref[...] * swcol, kb1_ref[...],
                             (((0,), (0,)), ((), ())),
                             preferred_element_type=jnp.float32)
        sw_ref[...] = (x0 + x1).reshape(262144)

    @pl.when(i == 0)
    def _():
        mn_ref[...] = jnp.full((1, 128), jnp.inf, jnp.float32)
        mx_ref[...] = jnp.full((1, 128), -jnp.inf, jnp.float32)

    w = x_ref[...].reshape(2048, 128) * sw_ref[...].reshape(2048, 128)
    mn_ref[...] = jnp.minimum(mn_ref[...], jnp.min(w))
    mx_ref[...] = jnp.maximum(mx_ref[...], jnp.max(w))


def _mlp_minmax_call(x1, pool3, W1, W2):
    return pl.pallas_call(
        _mlp_minmax_body,
        grid=(64,),
        in_specs=[
            pl.BlockSpec((262144,), lambda i: (i,)),
            pl.BlockSpec((1, 32, 64), lambda i: (i // 32, 0, 0)),
            pl.BlockSpec((4, 32), lambda i: (0, 0)),
            pl.BlockSpec((1, 4), lambda i: (0, 0)),
            pl.BlockSpec((64, 2048), lambda i: (0, 0)),
            pl.BlockSpec((64, 2048), lambda i: (0, 0)),
            pl.BlockSpec((64, 128), lambda i: (0, 0)),
            pl.BlockSpec((64, 128), lambda i: (0, 0)),
        ],
        out_specs=[
            pl.BlockSpec((262144,), lambda i: (i // 32,)),
            pl.BlockSpec((1, 128), lambda i: (0, 0)),
            pl.BlockSpec((1, 128), lambda i: (0, 0)),
        ],
        out_shape=[
            jax.ShapeDtypeStruct((524288,), jnp.float32),
            jax.ShapeDtypeStruct((1, 128), jnp.float32),
            jax.ShapeDtypeStruct((1, 128), jnp.float32),
        ],
    )(x1, pool3, W1, W2, jnp.asarray(_M0), jnp.asarray(_M1),
      jnp.asarray(_KB0), jnp.asarray(_KB1))


_CHUNK = 4 * 4096


def _sc_hist_body(x_hbm, sw_hbm, mm_hbm, out_hbm,
                  xb0, xb1, swb, mmb, hist, tileh, sem0, sem1):
    wid = lax.axis_index("s") * 2 + lax.axis_index("c")
    b = wid // 16
    d0 = (wid % 16) * 4

    pltpu.sync_copy(mm_hbm, mmb)
    gmin = mmb[pl.ds(0, 16)]
    gmax = mmb[pl.ds(128, 16)]
    scale = 128.0 / (gmax - gmin + 1e-8)
    gms = gmin * scale

    pltpu.sync_copy(sw_hbm.at[pl.ds(b * 262144 + d0 * 4096, _CHUNK)], swb)

    @plsc.parallel_loop(0, 1024, unroll=4)
    def _premul(j):
        swb[pl.ds(j * 16, 16)] = swb[pl.ds(j * 16, 16)] * scale

    zz = jnp.zeros((16,), jnp.float32)

    def _zh(j, carry):
        hist[pl.ds(j * 16, 16)] = zz
        return carry
    lax.fori_loop(0, 128, _zh, 0)

    laneoff = lax.iota(jnp.int32, 16) * 128
    aoff = laneoff.astype(jnp.float32) - gms
    clampv = laneoff + 127
    ones = jnp.ones((16,), jnp.float32)

    def _xoff(c):
        return ((b * 32 + c) * 64 + d0) * 4096

    bufs = (xb0, xb1)
    sems = (sem0, sem1)
    copies = [None, None]
    copies[0] = pltpu.async_copy(x_hbm.at[pl.ds(_xoff(0), _CHUNK)], xb0, sem0)
    for c in range(32):
        pb = c % 2
        if c + 1 < 32:
            nb = (c + 1) % 2
            copies[nb] = pltpu.async_copy(
                x_hbm.at[pl.ds(_xoff(c + 1), _CHUNK)], bufs[nb], sems[nb])
        copies[pb].wait()
        buf = bufs[pb]

        @plsc.parallel_loop(0, 1024, unroll=8)
        def _inner(j):
            off = j * 16
            xv = buf[pl.ds(off, 16)]
            sv = swb[pl.ds(off, 16)]
            bi = jnp.minimum((xv * sv + aoff).astype(jnp.int32), clampv)
            plsc.addupdate_scatter(hist, [bi], ones)

    for g in range(8):
        acc = hist[pl.ds(g * 16, 16)]
        for lane in range(1, 16):
            acc = acc + hist[pl.ds(lane * 128 + g * 16, 16)]
        tileh[pl.ds(g * 16, 16)] = acc
    pltpu.sync_copy(tileh, out_hbm.at[pl.ds(wid * 128, 128)])


def _sc_hist_call(x_flat, sw_flat, mm_flat):
    mesh = plsc.VectorSubcoreMesh(core_axis_name="c", subcore_axis_name="s")
    fn = pl.kernel(
        _sc_hist_body,
        mesh=mesh,
        compiler_params=pltpu.CompilerParams(needs_layout_passes=False),
        out_type=jax.ShapeDtypeStruct((4096,), jnp.float32),
        scratch_types=[
            pltpu.VMEM((_CHUNK,), jnp.float32),
            pltpu.VMEM((_CHUNK,), jnp.float32),
            pltpu.VMEM((_CHUNK,), jnp.float32),
            pltpu.VMEM((256,), jnp.float32),
            pltpu.VMEM((2048,), jnp.float32),
            pltpu.VMEM((128,), jnp.float32),
            pltpu.SemaphoreType.DMA,
            pltpu.SemaphoreType.DMA,
        ],
    )
    return fn(x_flat, sw_flat, mm_flat)


def _entropy_body(ph_ref, o_ref):
    h = jnp.sum(ph_ref[...], axis=0, keepdims=True)
    total = jnp.sum(h)
    prob = h / (total + 1e-10)
    ent = -jnp.sum(prob * jnp.log(prob + 1e-10)) * _LOG2E
    o_ref[...] = jnp.full((1, 1), ent, jnp.float32)


def _entropy_call(ph):
    return pl.pallas_call(
        _entropy_body,
        out_shape=jax.ShapeDtypeStruct((1, 1), jnp.float32),
    )(ph)


def kernel(x, W1, W2):
    x1 = x.reshape(-1)
    pool_p = _pool_call(x1)
    pool3 = pool_p.reshape(2, 32, 64)
    sw, mn, mx = _mlp_minmax_call(x1, pool3, W1, W2)
    mm = jnp.concatenate([mn, mx], axis=0).reshape(256)
    ph = _sc_hist_call(x1, sw, mm)
    ent = _entropy_call(ph.reshape(32, 128))
    return ent[0, 0]
